# baseline (device time: 141223 ns/iter reference)
import jax
import jax.numpy as jnp
from jax import lax
from jax.experimental import pallas as pl
from jax.experimental.pallas import tpu as pltpu

N_DEV = 4
BN = 2048


def kernel(x, w_mat, scale_x, scale_w):
    m_total, k_shard = x.shape
    k_total, n_total = w_mat.shape
    m_per = m_total // N_DEV
    n_blocks = n_total // BN

    my_i = lax.axis_index("i")
    rot = jnp.remainder(my_i + jnp.array([0, 3, 1, 2], jnp.int32), N_DEV)

    def body(rot_ref, x_ref, w_ref, sx_ref, sw_ref, out_ref,
             x_parts, xb16, send_sems, recv_sems):
        i = lax.axis_index("i")
        b = pl.program_id(0)
        t = pl.program_id(1)
        j = rot_ref[t]

        @pl.when((b == 0) & (t == 0))
        def _comm():
            barrier_sem = pltpu.get_barrier_semaphore()
            for off in range(1, N_DEV):
                peer = lax.rem(i + off, N_DEV)
                pl.semaphore_signal(
                    barrier_sem, inc=1,
                    device_id=(peer,), device_id_type=pl.DeviceIdType.MESH,
                )
            pl.semaphore_wait(barrier_sem, N_DEV - 1)

            for off in range(1, N_DEV):
                peer = lax.rem(i + off, N_DEV)
                rdma = pltpu.make_async_remote_copy(
                    src_ref=x_ref.at[pl.ds(peer * m_per, m_per), :],
                    dst_ref=x_parts.at[i],
                    send_sem=send_sems.at[off - 1],
                    recv_sem=recv_sems.at[i],
                    device_id=(peer,),
                    device_id_type=pl.DeviceIdType.MESH,
                )
                rdma.start()

            x_parts[i] = x_ref[pl.ds(i * m_per, m_per), :]

        @pl.when((b == 0) & (t > 0))
        def _wait_chunk():
            recv = pltpu.make_async_remote_copy(
                src_ref=x_parts.at[j],
                dst_ref=x_parts.at[j],
                send_sem=send_sems.at[0],
                recv_sem=recv_sems.at[j],
                device_id=(j,),
                device_id_type=pl.DeviceIdType.MESH,
            )
            recv.wait_recv()

        @pl.when(b == 0)
        def _convert():
            xb16[j] = x_parts[j].astype(jnp.bfloat16)

        d = lax.dot_general(
            xb16[j], w_ref[:, :].astype(jnp.bfloat16),
            (((1,), (0,)), ((), ())),
            preferred_element_type=jnp.float32,
        )

        @pl.when(t == 0)
        def _init():
            out_ref[:, :] = d

        @pl.when((t > 0) & (t < N_DEV - 1))
        def _accum():
            out_ref[:, :] = out_ref[:, :] + d

        @pl.when(t == N_DEV - 1)
        def _finish():
            y = (out_ref[:, :] + d) * (sx_ref[0] * sw_ref[0])
            yc = jnp.clip(y, -60.0, 60.0)
            out_ref[:, :] = y / (1.0 + jnp.exp(-yc))

        @pl.when((b == 0) & (t == N_DEV - 1))
        def _drain_sends():
            for off in range(1, N_DEV):
                send = pltpu.make_async_remote_copy(
                    src_ref=x_ref.at[pl.ds(0, m_per), :],
                    dst_ref=x_parts.at[i],
                    send_sem=send_sems.at[off - 1],
                    recv_sem=recv_sems.at[i],
                    device_id=(lax.rem(i + off, N_DEV),),
                    device_id_type=pl.DeviceIdType.MESH,
                )
                send.wait_send()

    return pl.pallas_call(
        body,
        grid_spec=pltpu.PrefetchScalarGridSpec(
            num_scalar_prefetch=1,
            grid=(n_blocks, N_DEV),
            in_specs=[
                pl.BlockSpec((m_total, k_shard), lambda b, t, rot: (0, 0)),
                pl.BlockSpec((k_shard, BN), lambda b, t, rot: (rot[t], b)),
                pl.BlockSpec(memory_space=pltpu.SMEM),
                pl.BlockSpec(memory_space=pltpu.SMEM),
            ],
            out_specs=pl.BlockSpec((m_per, BN), lambda b, t, rot: (0, b)),
            scratch_shapes=[
                pltpu.VMEM((N_DEV, m_per, k_shard), jnp.int8),
                pltpu.VMEM((N_DEV, m_per, k_shard), jnp.bfloat16),
                pltpu.SemaphoreType.DMA((N_DEV - 1,)),
                pltpu.SemaphoreType.DMA((N_DEV,)),
            ],
        ),
        out_shape=jax.ShapeDtypeStruct((m_per, n_total), jnp.float32),
        compiler_params=pltpu.CompilerParams(
            collective_id=0,
            dimension_semantics=("arbitrary", "arbitrary"),
            vmem_limit_bytes=40 * 1024 * 1024,
        ),
    )(rot, x, w_mat, scale_x, scale_w)
